# baseline (speedup 1.0000x reference)
import functools

import jax
import jax.numpy as jnp
from jax import lax
from jax.experimental import pallas as pl
from jax.experimental.pallas import tpu as pltpu
from jax.experimental.pallas import tpu_sc as plsc

_B = 16
_D = 1024
_L = 16


@functools.partial(
    pl.kernel,
    out_type=jax.ShapeDtypeStruct((_B,), jnp.float32),
    mesh=plsc.VectorSubcoreMesh(core_axis_name="c", subcore_axis_name="s",
                                num_cores=1),
    compiler_params=pltpu.CompilerParams(needs_layout_passes=False),
    scratch_types=[
        pltpu.VMEM((_B, 1), jnp.int32),
        pltpu.VMEM((_D,), jnp.float32),
        pltpu.VMEM((1, _D), jnp.float32),
        pltpu.VMEM((_L,), jnp.float32),
        pltpu.VMEM((_L,), jnp.float32),
        pltpu.SemaphoreType.DMA,
        pltpu.SemaphoreType.DMA,
        pltpu.SemaphoreType.DMA,
        pltpu.SemaphoreType.DMA,
    ],
)
def _sc_pool(hs_ref, w_ref, b_ref, cu_ref, out_ref,
             cu_v, w_v, piece_v, b_v, out_v, sem_w, sem_cu, sem_b, sem_g):
    c = lax.axis_index("c")
    s = lax.axis_index("s")

    @pl.when(c == 0)
    def _compute():
        lane = lax.iota(jnp.int32, _L)
        cp_w = pltpu.async_copy(w_ref, w_v, sem_w)
        cp_cu = pltpu.async_copy(cu_ref, cu_v, sem_cu)
        cp_b = pltpu.async_copy(b_ref.at[jnp.zeros((_L,), jnp.int32)], b_v,
                                sem_b)
        cp_cu.wait()
        pltpu.async_copy(hs_ref.at[cu_v.at[s]], piece_v, sem_g).wait()
        cp_w.wait()
        acc = jnp.zeros((_L,), dtype=jnp.float32)
        for i in range(_D // _L):
            acc = acc + piece_v[0, pl.ds(i * _L, _L)] * w_v[pl.ds(i * _L, _L)]
        logit = jnp.sum(acc)
        cp_b.wait()
        score = 1.0 / (1.0 + jnp.exp(-(logit + b_v[...])))
        out_v[...] = score
        pltpu.async_copy(out_v, out_ref.at[jnp.broadcast_to(s, (_L,))],
                         sem_g).wait()


def kernel(hidden_states, W, b, cu_seqlens):
    w1 = W.reshape(_D)
    starts2 = cu_seqlens[:-1].reshape(_B, 1)
    return _sc_pool(hidden_states, w1, b, starts2)
